# baseline (speedup 1.0000x reference)
import functools

import jax
import jax.numpy as jnp
from jax import lax
from jax.experimental import pallas as pl
from jax.experimental.pallas import tpu as pltpu
from jax.experimental.pallas import tpu_sc as plsc

N = 10000
N_PAD = 10240
E = 320000
CHUNK = 128
NB = 80
E_PAD = 327680
NCHUNK = 2560
NB0 = 48
NB1 = 112
BLK = 8
ROWS_T = 640
D_IN = 128
H1 = 256
H2 = 128


def _deg_body(src_hbm, dst_hbm, zvec_hbm, dout_hbm, din_hbm,
              src_v, dst_v, ones_v, dout_sh, din_sh, sem1, sem2):
    c = lax.axis_index("c")
    s = lax.axis_index("s")
    wid = c * 16 + s
    row = pl.ds(s * ROWS_T, ROWS_T)
    pltpu.sync_copy(zvec_hbm.at[row], dout_sh.at[row])
    pltpu.sync_copy(zvec_hbm.at[row], din_sh.at[row])
    pltpu.sync_copy(src_hbm.at[wid], src_v)
    pltpu.sync_copy(dst_hbm.at[wid], dst_v)
    for i in range(8):
        ones_v[pl.ds(i * 16, 16)] = jnp.full((16,), 1.0, jnp.float32)
    plsc.subcore_barrier()

    def fire(j, carry):
        pltpu.async_copy(ones_v, dout_sh.at[src_v.at[j]], sem1, add=True)
        pltpu.async_copy(ones_v, din_sh.at[dst_v.at[j]], sem2, add=True)
        return carry

    lax.fori_loop(0, NB, fire, 0)

    def drain(j, carry):
        pltpu.make_async_copy(ones_v, dout_sh.at[src_v.at[j]], sem1).wait()
        pltpu.make_async_copy(ones_v, din_sh.at[dst_v.at[j]], sem2).wait()
        return carry

    lax.fori_loop(0, NB, drain, 0)
    plsc.subcore_barrier()
    pltpu.sync_copy(dout_sh.at[row], dout_hbm.at[wid])
    pltpu.sync_copy(din_sh.at[row], din_hbm.at[wid])


def _edge_body(xp_hbm, nd_hbm, ei_hbm, zrow_hbm, zvec_hbm,
               agg_hbm, c_hbm,
               iblk, rows_a, rows_b, vals_v, agg_sh, c_sh,
               semi_a, semi_b, semg_a, semg_b, semv_a, semv_b,
               sems_a, sems_b, semw):
    c = lax.axis_index("c")
    s = lax.axis_index("s")
    wid = c * 16 + s
    row = pl.ds(s * ROWS_T, ROWS_T)
    pltpu.sync_copy(zrow_hbm.at[row], agg_sh.at[row])
    pltpu.sync_copy(zvec_hbm.at[row], c_sh.at[row])
    plsc.subcore_barrier()

    base = jnp.where(c == 0, s * NB0, 16 * NB0 + s * NB1)
    nblk = jnp.where(c == 0, NB0 // BLK, NB1 // BLK)

    rows = (rows_a, rows_b)
    semi = (semi_a, semi_b)
    semg = (semg_a, semg_b)
    semv = (semv_a, semv_b)
    sems = (sems_a, sems_b)

    def ld_idx(bi, slot):
        pltpu.async_copy(ei_hbm.at[pl.ds(base + bi * BLK, BLK)],
                         iblk.at[slot], semi[slot])

    def w_idx(slot):
        pltpu.make_async_copy(ei_hbm.at[pl.ds(0, BLK)], iblk.at[slot],
                              semi[slot]).wait()

    def g_rows(slot, j):
        pltpu.async_copy(xp_hbm.at[iblk.at[slot, j, 0]], rows[j % 2],
                         semg[j % 2])

    def w_g(j):
        pltpu.make_async_copy(xp_hbm.at[iblk.at[0, 0, 0]], rows[j % 2],
                              semg[j % 2]).wait()

    def g_vals(slot, j):
        pltpu.async_copy(nd_hbm.at[iblk.at[slot, j, 1]], vals_v.at[slot, j],
                         semv[j % 2])

    def w_v(j):
        pltpu.make_async_copy(nd_hbm.at[iblk.at[0, 0, 1]], vals_v.at[0, 0],
                              semv[j % 2]).wait()

    def s_rows(slot, j):
        pltpu.async_copy(rows[j % 2], agg_sh.at[iblk.at[slot, j, 1]],
                         sems[j % 2], add=True)

    def w_s(j):
        pltpu.make_async_copy(rows[j % 2], agg_sh.at[iblk.at[0, 0, 1]],
                              sems[j % 2]).wait()

    def s_vals(slot, j):
        pltpu.async_copy(vals_v.at[slot, j], c_sh.at[iblk.at[slot, j, 0]],
                         semw, add=True)

    def w_w():
        pltpu.make_async_copy(vals_v.at[0, 0], c_sh.at[iblk.at[0, 0, 0]],
                              semw).wait()

    def process_block(slot):
        w_idx(slot)
        g_rows(slot, 0)
        g_vals(slot, 0)
        g_rows(slot, 1)
        g_vals(slot, 1)
        for j in range(BLK):
            w_g(j)
            s_rows(slot, j)
            w_v(j)
            s_vals(slot, j)
            if j < BLK - 2:
                w_s(j)
                g_rows(slot, j + 2)
                g_vals(slot, j + 2)
        w_s(BLK - 2)
        w_s(BLK - 1)
        for _ in range(BLK):
            w_w()

    ld_idx(0, 0)
    ld_idx(1, 1)

    def body(i2, carry):
        bi0 = 2 * i2

        process_block(0)

        @pl.when(bi0 + 2 < nblk)
        def _():
            ld_idx(bi0 + 2, 0)

        process_block(1)

        @pl.when(bi0 + 3 < nblk)
        def _():
            ld_idx(bi0 + 3, 1)

        return carry

    lax.fori_loop(0, nblk // 2, body, 0)
    plsc.subcore_barrier()
    pltpu.sync_copy(agg_sh.at[row], agg_hbm.at[wid])
    pltpu.sync_copy(c_sh.at[row], c_hbm.at[wid])


@functools.cache
def _sc_kernels():
    mesh = plsc.VectorSubcoreMesh(core_axis_name="c", subcore_axis_name="s",
                                  num_cores=2, num_subcores=16)
    deg = pl.kernel(
        _deg_body,
        out_type=(
            jax.ShapeDtypeStruct((32, ROWS_T), jnp.float32),
            jax.ShapeDtypeStruct((32, ROWS_T), jnp.float32),
        ),
        mesh=mesh,
        scratch_types=[
            pltpu.VMEM((NB, 128), jnp.int32),
            pltpu.VMEM((NB, 128), jnp.int32),
            pltpu.VMEM((128,), jnp.float32),
            pltpu.VMEM_SHARED((N_PAD,), jnp.float32),
            pltpu.VMEM_SHARED((N_PAD,), jnp.float32),
            pltpu.SemaphoreType.DMA,
            pltpu.SemaphoreType.DMA,
        ],
    )
    edge = pl.kernel(
        _edge_body,
        out_type=(
            jax.ShapeDtypeStruct((32, ROWS_T, D_IN), jnp.float32),
            jax.ShapeDtypeStruct((32, ROWS_T), jnp.float32),
        ),
        mesh=mesh,
        scratch_types=[
            pltpu.VMEM((2, BLK, 2, 128), jnp.int32),
            pltpu.VMEM((CHUNK, D_IN), jnp.float32),
            pltpu.VMEM((CHUNK, D_IN), jnp.float32),
            pltpu.VMEM((2, BLK, 128), jnp.float32),
            pltpu.VMEM_SHARED((N_PAD, D_IN), jnp.float32),
            pltpu.VMEM_SHARED((N_PAD,), jnp.float32),
        ] + [pltpu.SemaphoreType.DMA] * 9,
    )
    return deg, edge


def _prep_body(x_ref, dop_ref, dip_ref, ns_ref, nd_ref, xp_ref):
    do = dop_ref[0] + dop_ref[1]
    di = dip_ref[0] + dip_ref[1]
    ns = lax.rsqrt(jnp.maximum(do, 1.0))
    nd = lax.rsqrt(jnp.maximum(di, 1.0))
    ns_ref[...] = ns
    nd_ref[...] = nd
    xp_ref[...] = jnp.where(
        lax.broadcasted_iota(jnp.int32, (N_PAD, 1), 0) < N,
        jnp.pad(x_ref[...], ((0, N_PAD - N), (0, 0))) * ns, 0.0)


_prep = pl.pallas_call(
    _prep_body,
    out_shape=(
        jax.ShapeDtypeStruct((N_PAD, 1), jnp.float32),
        jax.ShapeDtypeStruct((N_PAD, 1), jnp.float32),
        jax.ShapeDtypeStruct((N_PAD, D_IN), jnp.float32),
    ),
)


def _final_body(aggp_ref, cp_ref, nd_ref, ns_ref, w1_ref, b1_ref, w2_ref,
                b2_ref, out_ref):
    agg = (aggp_ref[0] + aggp_ref[1]) * nd_ref[...]
    h = jnp.dot(agg, w1_ref[...], preferred_element_type=jnp.float32)
    q = jnp.maximum(h + b1_ref[...], 0.0) * ns_ref[...]
    rid = lax.broadcasted_iota(jnp.int32, (N_PAD, 1), 0)
    q = jnp.where(rid < N, q, 0.0)
    cvec = cp_ref[0] + cp_ref[1]
    cvec = jnp.where(rid < N, cvec, 0.0)
    v = jnp.sum(q * cvec, axis=0, keepdims=True)
    out_ref[...] = (jnp.dot(v, w2_ref[...], preferred_element_type=jnp.float32)
                    * (1.0 / N) + b2_ref[...])


_final = pl.pallas_call(
    _final_body,
    out_shape=jax.ShapeDtypeStruct((1, H2), jnp.float32),
)


def kernel(x, edge_index, W1, b1, W2, b2):
    src = edge_index[0].astype(jnp.int32)
    dst = edge_index[1].astype(jnp.int32)
    pad = E_PAD - E
    padv = jnp.full((pad,), N, jnp.int32)
    src_p = jnp.concatenate([src, padv]).reshape(32, NB, 128)
    dst_p = jnp.concatenate([dst, padv]).reshape(32, NB, 128)
    ei_p = jnp.stack([src_p.reshape(NCHUNK, 128),
                      dst_p.reshape(NCHUNK, 128)], axis=1)
    zvec = jnp.zeros((N_PAD,), jnp.float32)
    zrow = jnp.zeros((N_PAD, D_IN), jnp.float32)

    deg_kernel, edge_kernel = _sc_kernels()
    dout_t, din_t = deg_kernel(src_p, dst_p, zvec)
    ns, nd, xp = _prep(x,
                       dout_t.reshape(2, N_PAD, 1),
                       din_t.reshape(2, N_PAD, 1))
    agg_t, c_t = edge_kernel(xp, nd.reshape(N_PAD), ei_p, zrow, zvec)
    out = _final(agg_t.reshape(2, N_PAD, D_IN), c_t.reshape(2, N_PAD, 1),
                 nd, ns, W1, b1.reshape(1, H1), W2, b2.reshape(1, H2))
    return out


# speedup vs baseline: 1.2775x; 1.2775x over previous
import functools

import jax
import jax.numpy as jnp
from jax import lax
from jax.experimental import pallas as pl
from jax.experimental.pallas import tpu as pltpu
from jax.experimental.pallas import tpu_sc as plsc

N = 10000
N_PAD = 10240
E = 320000
CHUNK = 128
NB = 80
E_PAD = 327680
NCHUNK = 2560
NB0 = 48
NB1 = 112
BLK = 8
ROWS_T = 640
D_IN = 128
H1 = 256
H2 = 128


def _deg_body(src_hbm, dst_hbm, zvec_hbm, dout_hbm, din_hbm,
              src_v, dst_v, ones_v, dout_sh, din_sh, sem1, sem2):
    c = lax.axis_index("c")
    s = lax.axis_index("s")
    wid = c * 16 + s
    row = pl.ds(s * ROWS_T, ROWS_T)
    pltpu.sync_copy(zvec_hbm.at[row], dout_sh.at[row])
    pltpu.sync_copy(zvec_hbm.at[row], din_sh.at[row])
    pltpu.sync_copy(src_hbm.at[wid], src_v)
    pltpu.sync_copy(dst_hbm.at[wid], dst_v)
    for i in range(8):
        ones_v[pl.ds(i * 16, 16)] = jnp.full((16,), 1.0, jnp.float32)
    plsc.subcore_barrier()

    def fire(j, carry):
        pltpu.async_copy(ones_v, dout_sh.at[src_v.at[j]], sem1, add=True)
        pltpu.async_copy(ones_v, din_sh.at[dst_v.at[j]], sem2, add=True)
        return carry

    lax.fori_loop(0, NB, fire, 0)

    def drain(j, carry):
        pltpu.make_async_copy(ones_v, dout_sh.at[src_v.at[j]], sem1).wait()
        pltpu.make_async_copy(ones_v, din_sh.at[dst_v.at[j]], sem2).wait()
        return carry

    lax.fori_loop(0, NB, drain, 0)
    plsc.subcore_barrier()
    pltpu.sync_copy(dout_sh.at[row], dout_hbm.at[wid])
    pltpu.sync_copy(din_sh.at[row], din_hbm.at[wid])


def _edge_body(xp_hbm, nd_hbm, src_hbm, dst_hbm, zrow_hbm, zvec_hbm,
               agg_hbm, c_hbm,
               src_v, dst_v, rows_v, vals_v, agg_sh, c_sh, sem1, sem2):
    c = lax.axis_index("c")
    s = lax.axis_index("s")
    wid = c * 16 + s
    row = pl.ds(s * ROWS_T, ROWS_T)
    pltpu.sync_copy(zrow_hbm.at[row], agg_sh.at[row])
    pltpu.sync_copy(zvec_hbm.at[row], c_sh.at[row])
    pltpu.sync_copy(src_hbm.at[wid], src_v)
    pltpu.sync_copy(dst_hbm.at[wid], dst_v)
    plsc.subcore_barrier()

    def body(j, carry):
        g1 = pltpu.async_copy(xp_hbm.at[src_v.at[j]], rows_v, sem1)
        g2 = pltpu.async_copy(nd_hbm.at[dst_v.at[j]], vals_v, sem2)
        g1.wait()
        g2.wait()
        pltpu.sync_copy(rows_v, agg_sh.at[dst_v.at[j]], add=True)
        pltpu.sync_copy(vals_v, c_sh.at[src_v.at[j]], add=True)
        return carry

    lax.fori_loop(0, NB, body, 0)
    plsc.subcore_barrier()
    pltpu.sync_copy(agg_sh.at[row], agg_hbm.at[wid])
    pltpu.sync_copy(c_sh.at[row], c_hbm.at[wid])


@functools.cache
def _sc_kernels():
    mesh = plsc.VectorSubcoreMesh(core_axis_name="c", subcore_axis_name="s",
                                  num_cores=2, num_subcores=16)
    deg = pl.kernel(
        _deg_body,
        out_type=(
            jax.ShapeDtypeStruct((32, ROWS_T), jnp.float32),
            jax.ShapeDtypeStruct((32, ROWS_T), jnp.float32),
        ),
        mesh=mesh,
        scratch_types=[
            pltpu.VMEM((NB, 128), jnp.int32),
            pltpu.VMEM((NB, 128), jnp.int32),
            pltpu.VMEM((128,), jnp.float32),
            pltpu.VMEM_SHARED((N_PAD,), jnp.float32),
            pltpu.VMEM_SHARED((N_PAD,), jnp.float32),
            pltpu.SemaphoreType.DMA,
            pltpu.SemaphoreType.DMA,
        ],
    )
    edge = pl.kernel(
        _edge_body,
        out_type=(
            jax.ShapeDtypeStruct((32, ROWS_T, D_IN), jnp.float32),
            jax.ShapeDtypeStruct((32, ROWS_T), jnp.float32),
        ),
        mesh=mesh,
        scratch_types=[
            pltpu.VMEM((NB, 128), jnp.int32),
            pltpu.VMEM((NB, 128), jnp.int32),
            pltpu.VMEM((CHUNK, D_IN), jnp.float32),
            pltpu.VMEM((128,), jnp.float32),
            pltpu.VMEM_SHARED((N_PAD, D_IN), jnp.float32),
            pltpu.VMEM_SHARED((N_PAD,), jnp.float32),
            pltpu.SemaphoreType.DMA,
            pltpu.SemaphoreType.DMA,
        ],
    )
    return deg, edge


def _prep_body(x_ref, dop_ref, dip_ref, ns_ref, nd_ref, xp_ref):
    do = dop_ref[0] + dop_ref[1]
    di = dip_ref[0] + dip_ref[1]
    ns = lax.rsqrt(jnp.maximum(do, 1.0))
    nd = lax.rsqrt(jnp.maximum(di, 1.0))
    ns_ref[...] = ns
    nd_ref[...] = nd
    xp_ref[...] = jnp.where(
        lax.broadcasted_iota(jnp.int32, (N_PAD, 1), 0) < N,
        jnp.pad(x_ref[...], ((0, N_PAD - N), (0, 0))) * ns, 0.0)


_prep = pl.pallas_call(
    _prep_body,
    out_shape=(
        jax.ShapeDtypeStruct((N_PAD, 1), jnp.float32),
        jax.ShapeDtypeStruct((N_PAD, 1), jnp.float32),
        jax.ShapeDtypeStruct((N_PAD, D_IN), jnp.float32),
    ),
)


def _final_body(aggp_ref, cp_ref, nd_ref, ns_ref, w1_ref, b1_ref, w2_ref,
                b2_ref, out_ref):
    agg = (aggp_ref[0] + aggp_ref[1]) * nd_ref[...]
    h = jnp.dot(agg, w1_ref[...], preferred_element_type=jnp.float32)
    q = jnp.maximum(h + b1_ref[...], 0.0) * ns_ref[...]
    rid = lax.broadcasted_iota(jnp.int32, (N_PAD, 1), 0)
    q = jnp.where(rid < N, q, 0.0)
    cvec = cp_ref[0] + cp_ref[1]
    cvec = jnp.where(rid < N, cvec, 0.0)
    v = jnp.sum(q * cvec, axis=0, keepdims=True)
    out_ref[...] = (jnp.dot(v, w2_ref[...], preferred_element_type=jnp.float32)
                    * (1.0 / N) + b2_ref[...])


_final = pl.pallas_call(
    _final_body,
    out_shape=jax.ShapeDtypeStruct((1, H2), jnp.float32),
)


def kernel(x, edge_index, W1, b1, W2, b2):
    src = edge_index[0].astype(jnp.int32)
    dst = edge_index[1].astype(jnp.int32)
    pad = E_PAD - E
    padv = jnp.full((pad,), N, jnp.int32)
    src_p = jnp.concatenate([src, padv]).reshape(32, NB, 128)
    dst_p = jnp.concatenate([dst, padv]).reshape(32, NB, 128)
    zvec = jnp.zeros((N_PAD,), jnp.float32)
    zrow = jnp.zeros((N_PAD, D_IN), jnp.float32)

    deg_kernel, edge_kernel = _sc_kernels()
    dout_t, din_t = deg_kernel(src_p, dst_p, zvec)
    ns, nd, xp = _prep(x,
                       dout_t.reshape(2, N_PAD, 1),
                       din_t.reshape(2, N_PAD, 1))
    agg_t, c_t = edge_kernel(xp, nd.reshape(N_PAD), src_p, dst_p, zrow, zvec)
    out = _final(agg_t.reshape(2, N_PAD, D_IN), c_t.reshape(2, N_PAD, 1),
                 nd, ns, W1, b1.reshape(1, H1), W2, b2.reshape(1, H2))
    return out


# speedup vs baseline: 1.5902x; 1.2448x over previous
import functools

import jax
import jax.numpy as jnp
from jax import lax
from jax.experimental import pallas as pl
from jax.experimental.pallas import tpu as pltpu
from jax.experimental.pallas import tpu_sc as plsc

N = 10000
N_PAD = 10240
E = 320000
CH = 79
E_T = CH * 128
E_PAD = 32 * E_T
ROWS_T = 640
D_IN = 128
H1 = 256
H2 = 128


def _deg_body(src_hbm, dst_hbm, zvec_hbm, dout_hbm, din_hbm,
              src_v, dst_v, ones_v, dout_sh, din_sh):
    c = lax.axis_index("c")
    s = lax.axis_index("s")
    wid = c * 16 + s
    row = pl.ds(s * ROWS_T, ROWS_T)
    pltpu.sync_copy(zvec_hbm.at[row], dout_sh.at[row])
    pltpu.sync_copy(zvec_hbm.at[row], din_sh.at[row])
    pltpu.sync_copy(src_hbm.at[wid], src_v)
    pltpu.sync_copy(dst_hbm.at[wid], dst_v)
    for i in range(8):
        ones_v[pl.ds(i * 16, 16)] = jnp.full((16,), 1.0, jnp.float32)
    plsc.subcore_barrier()

    def body(j, carry):
        pltpu.sync_copy(ones_v, dout_sh.at[src_v.at[j]], add=True)
        pltpu.sync_copy(ones_v, din_sh.at[dst_v.at[j]], add=True)
        return carry

    lax.fori_loop(0, CH, body, 0)
    plsc.subcore_barrier()
    pltpu.sync_copy(dout_sh.at[row], dout_hbm.at[wid])
    pltpu.sync_copy(din_sh.at[row], din_hbm.at[wid])


def _edge_body(xp_hbm, nd_hbm, src_hbm, dst_hbm, zrow_hbm, zvec_hbm,
               agg_hbm, c_hbm,
               src_v, dst_v, rows_v, vals_v, agg_sh, c_sh, sem1, sem2):
    c = lax.axis_index("c")
    s = lax.axis_index("s")
    wid = c * 16 + s
    row = pl.ds(s * ROWS_T, ROWS_T)
    pltpu.sync_copy(zrow_hbm.at[row], agg_sh.at[row])
    pltpu.sync_copy(zvec_hbm.at[row], c_sh.at[row])
    pltpu.sync_copy(src_hbm.at[wid], src_v)
    pltpu.sync_copy(dst_hbm.at[wid], dst_v)
    plsc.subcore_barrier()

    def body(j, carry):
        g1 = pltpu.async_copy(xp_hbm.at[src_v.at[j]], rows_v, sem1)
        g2 = pltpu.async_copy(nd_hbm.at[dst_v.at[j]], vals_v, sem2)
        g1.wait()
        g2.wait()
        pltpu.sync_copy(rows_v, agg_sh.at[dst_v.at[j]], add=True)
        pltpu.sync_copy(vals_v, c_sh.at[src_v.at[j]], add=True)
        return carry

    lax.fori_loop(0, CH, body, 0)
    plsc.subcore_barrier()
    pltpu.sync_copy(agg_sh.at[row], agg_hbm.at[wid])
    pltpu.sync_copy(c_sh.at[row], c_hbm.at[wid])


@functools.cache
def _sc_kernels():
    mesh = plsc.VectorSubcoreMesh(core_axis_name="c", subcore_axis_name="s",
                                  num_cores=2, num_subcores=16)
    deg = pl.kernel(
        _deg_body,
        out_type=(
            jax.ShapeDtypeStruct((32, ROWS_T), jnp.float32),
            jax.ShapeDtypeStruct((32, ROWS_T), jnp.float32),
        ),
        mesh=mesh,
        scratch_types=[
            pltpu.VMEM((CH, 128), jnp.int32),
            pltpu.VMEM((CH, 128), jnp.int32),
            pltpu.VMEM((128,), jnp.float32),
            pltpu.VMEM_SHARED((N_PAD,), jnp.float32),
            pltpu.VMEM_SHARED((N_PAD,), jnp.float32),
        ],
    )
    edge = pl.kernel(
        _edge_body,
        out_type=(
            jax.ShapeDtypeStruct((32, ROWS_T, D_IN), jnp.float32),
            jax.ShapeDtypeStruct((32, ROWS_T), jnp.float32),
        ),
        mesh=mesh,
        scratch_types=[
            pltpu.VMEM((CH, 128), jnp.int32),
            pltpu.VMEM((CH, 128), jnp.int32),
            pltpu.VMEM((128, D_IN), jnp.float32),
            pltpu.VMEM((128,), jnp.float32),
            pltpu.VMEM_SHARED((N_PAD, D_IN), jnp.float32),
            pltpu.VMEM_SHARED((N_PAD,), jnp.float32),
            pltpu.SemaphoreType.DMA,
            pltpu.SemaphoreType.DMA,
        ],
    )
    return deg, edge


def _prep_body(x_ref, dop_ref, dip_ref, ns_ref, nd_ref, xp_ref):
    do = dop_ref[0] + dop_ref[1]
    di = dip_ref[0] + dip_ref[1]
    ns = lax.rsqrt(jnp.maximum(do, 1.0))
    nd = lax.rsqrt(jnp.maximum(di, 1.0))
    ns_ref[...] = ns
    nd_ref[...] = nd
    xp_ref[...] = x_ref[...] * ns


_prep = pl.pallas_call(
    _prep_body,
    out_shape=(
        jax.ShapeDtypeStruct((N_PAD, 1), jnp.float32),
        jax.ShapeDtypeStruct((N_PAD, 1), jnp.float32),
        jax.ShapeDtypeStruct((N_PAD, D_IN), jnp.float32),
    ),
)


def _final_body(aggp_ref, cp_ref, nd_ref, ns_ref, w1_ref, b1_ref, w2_ref,
                b2_ref, out_ref):
    agg = (aggp_ref[0] + aggp_ref[1]) * nd_ref[...]
    h = jnp.dot(agg, w1_ref[...], preferred_element_type=jnp.float32)
    q = jnp.maximum(h + b1_ref[...], 0.0) * ns_ref[...]
    cvec = cp_ref[0] + cp_ref[1]
    rid = lax.broadcasted_iota(jnp.int32, (N_PAD, 1), 0)
    cvec = jnp.where(rid < N, cvec, 0.0)
    v = jnp.sum(q * cvec, axis=0, keepdims=True)
    out_ref[...] = (jnp.dot(v, w2_ref[...], preferred_element_type=jnp.float32)
                    * (1.0 / N) + b2_ref[...])


_final = pl.pallas_call(
    _final_body,
    out_shape=jax.ShapeDtypeStruct((1, H2), jnp.float32),
)


def kernel(x, edge_index, W1, b1, W2, b2):
    src = edge_index[0].astype(jnp.int32)
    dst = edge_index[1].astype(jnp.int32)
    pad = E_PAD - E
    padv = jnp.full((pad,), N, jnp.int32)
    src_p = jnp.concatenate([src, padv]).reshape(32, CH, 128)
    dst_p = jnp.concatenate([dst, padv]).reshape(32, CH, 128)
    x_pad = jnp.zeros((N_PAD, D_IN), jnp.float32).at[:N].set(x)
    zvec = jnp.zeros((N_PAD,), jnp.float32)
    zrow = jnp.zeros((N_PAD, D_IN), jnp.float32)

    deg_kernel, edge_kernel = _sc_kernels()
    dout_t, din_t = deg_kernel(src_p, dst_p, zvec)
    ns, nd, xp = _prep(x_pad,
                       dout_t.reshape(2, N_PAD, 1),
                       din_t.reshape(2, N_PAD, 1))
    agg_t, c_t = edge_kernel(xp, nd.reshape(N_PAD), src_p, dst_p, zrow, zvec)
    out = _final(agg_t.reshape(2, N_PAD, D_IN), c_t.reshape(2, N_PAD, 1),
                 nd, ns, W1, b1.reshape(1, H1), W2, b2.reshape(1, H2))
    return out
